# baseline (speedup 1.0000x reference)
import functools

import jax
import jax.numpy as jnp
from jax import lax
from jax.experimental import pallas as pl
from jax.experimental.pallas import tpu as pltpu
from jax.experimental.pallas import tpu_sc as plsc

E = 8
K = 2
D = 768
F = 2048
ALPHA = 0.1
TS = 128
NP = 4096 + E * TS
NT = NP // TS


def _router_kernel(x_ref, wg_ref, i1_ref, i2_ref, w1_ref, w2_ref,
                   r1_ref, r2_ref, ce_ref, aux_ref, ce_acc, ss_acc, *, T, TM):
    i = pl.program_id(0)
    nt = pl.num_programs(0)
    x = x_ref[...]
    logits = jax.lax.dot_general(
        x, wg_ref[...], (((1,), (1,)), ((), ())),
        preferred_element_type=jnp.float32)
    m = jnp.max(logits, axis=1, keepdims=True)
    ex = jnp.exp(logits - m)
    scores = ex / jnp.sum(ex, axis=1, keepdims=True)

    lane = jax.lax.broadcasted_iota(jnp.int32, scores.shape, 1)
    s1 = jnp.max(scores, axis=1, keepdims=True)
    i1 = jnp.min(jnp.where(scores == s1, lane, E), axis=1, keepdims=True)
    masked = jnp.where(lane == i1, -jnp.inf, scores)
    s2 = jnp.max(masked, axis=1, keepdims=True)
    i2 = jnp.min(jnp.where(masked == s2, lane, E), axis=1, keepdims=True)
    denom = s1 + s2 + 1e-20
    oh1 = (lane == i1).astype(jnp.float32)
    oh2 = (lane == i2).astype(jnp.float32)

    i1_ref[...] = i1
    i2_ref[...] = i2
    w1_ref[...] = s1 / denom
    w2_ref[...] = s2 / denom

    @pl.when(i == 0)
    def _():
        ce_acc[...] = jnp.zeros_like(ce_acc)
        ss_acc[...] = jnp.zeros_like(ss_acc)

    cnt = oh1 + oh2
    row = jax.lax.broadcasted_iota(jnp.int32, (TM, TM), 0)
    col = jax.lax.broadcasted_iota(jnp.int32, (TM, TM), 1)
    lstrict = (col < row).astype(jnp.float32)
    pref = jax.lax.dot_general(
        lstrict, cnt, (((1,), (0,)), ((), ())),
        preferred_element_type=jnp.float32,
        precision=jax.lax.Precision.HIGHEST)
    pref = pref + ce_acc[...]
    r1_ref[...] = jnp.sum(pref * oh1, axis=1, keepdims=True).astype(jnp.int32)
    r2_ref[...] = jnp.sum(pref * oh2, axis=1, keepdims=True).astype(jnp.int32)

    ce_acc[...] += jnp.sum(cnt, axis=0, keepdims=True)
    ss_acc[...] += jnp.sum(scores, axis=0, keepdims=True)

    @pl.when(i == nt - 1)
    def _():
        ce_ref[...] = ce_acc[...].astype(jnp.int32)
        ce = ce_acc[...] / (T * K / E)
        aux_ref[...] = jnp.sum(ce * (ss_acc[...] / T), keepdims=True).reshape(1, 1) * ALPHA


def _pos_kernel(i1_ref, i2_ref, r1_ref, r2_ref, poff_ref, pos1_ref, pos2_ref,
                tok_ref, *, TM):
    i = pl.program_id(0)
    lane = jax.lax.broadcasted_iota(jnp.int32, (TM, E), 1)
    poff = poff_ref[...]
    b1 = jnp.sum(jnp.where(lane == i1_ref[...], poff, 0), axis=1, keepdims=True)
    b2 = jnp.sum(jnp.where(lane == i2_ref[...], poff, 0), axis=1, keepdims=True)
    pos1_ref[...] = b1 + r1_ref[...]
    pos2_ref[...] = b2 + r2_ref[...]
    tok_ref[...] = i * TM + jax.lax.broadcasted_iota(jnp.int32, (TM, 1), 0)


def _make_dispatch(T):
    info = plsc.get_sparse_core_info()
    NC, NS, L = info.num_cores, info.num_subcores, info.num_lanes
    NW = NC * NS
    BW = (T * K) // NW
    mesh = plsc.VectorSubcoreMesh(core_axis_name="c", subcore_axis_name="s")

    @functools.partial(
        pl.kernel, mesh=mesh,
        out_type=jax.ShapeDtypeStruct((NP, D), jnp.float32),
        scratch_types=[
            pltpu.VMEM((BW,), jnp.int32),
            pltpu.VMEM((BW,), jnp.int32),
            pltpu.VMEM((BW, D), jnp.float32),
            pltpu.SemaphoreType.DMA,
        ],
    )
    def dispatch(xf_hbm, tokflat_hbm, posflat_hbm, xg_hbm, tok_v, pos_v, rows_v, sem):
        wid = lax.axis_index("s") * NC + lax.axis_index("c")
        base = wid * BW
        pltpu.sync_copy(tokflat_hbm.at[pl.ds(base, BW)], tok_v)
        pltpu.sync_copy(posflat_hbm.at[pl.ds(base, BW)], pos_v)
        pltpu.async_copy(xf_hbm.at[tok_v], rows_v, sem).wait()
        pltpu.async_copy(rows_v, xg_hbm.at[pos_v], sem).wait()

    return dispatch


def _make_return(T):
    info = plsc.get_sparse_core_info()
    NC, NS, L = info.num_cores, info.num_subcores, info.num_lanes
    NW = NC * NS
    BW = T // NW
    mesh = plsc.VectorSubcoreMesh(core_axis_name="c", subcore_axis_name="s")

    @functools.partial(
        pl.kernel, mesh=mesh,
        out_type=[jax.ShapeDtypeStruct((T, D), jnp.float32),
                  jax.ShapeDtypeStruct((T, D), jnp.float32)],
        scratch_types=[
            pltpu.VMEM((BW,), jnp.int32),
            pltpu.VMEM((BW, D), jnp.float32),
            pltpu.SemaphoreType.DMA,
        ],
    )
    def ret(eo_hbm, pos1_hbm, pos2_hbm, g1_hbm, g2_hbm, idx_v, rows_v, sem):
        wid = lax.axis_index("s") * NC + lax.axis_index("c")
        base = wid * BW
        pltpu.sync_copy(pos1_hbm.at[pl.ds(base, BW)], idx_v)
        pltpu.async_copy(eo_hbm.at[idx_v], rows_v, sem).wait()
        pltpu.sync_copy(rows_v, g1_hbm.at[pl.ds(base, BW)])
        pltpu.sync_copy(pos2_hbm.at[pl.ds(base, BW)], idx_v)
        pltpu.async_copy(eo_hbm.at[idx_v], rows_v, sem).wait()
        pltpu.sync_copy(rows_v, g2_hbm.at[pl.ds(base, BW)])

    return ret


def _ffn_kernel(te_ref, xg_ref, w1_ref, w3_ref, w2_ref, eo_ref):
    xg = xg_ref[...]
    h1 = jax.lax.dot_general(
        xg, w1_ref[0], (((1,), (1,)), ((), ())), preferred_element_type=jnp.float32)
    h3 = jax.lax.dot_general(
        xg, w3_ref[0], (((1,), (1,)), ((), ())), preferred_element_type=jnp.float32)
    act = h1 * jax.nn.sigmoid(h1) * h3
    eo_ref[...] = jax.lax.dot_general(
        act, w2_ref[0], (((1,), (1,)), ((), ())), preferred_element_type=jnp.float32)


def _combine_kernel(w1_ref, w2_ref, g1_ref, g2_ref, y_ref):
    y_ref[...] = w1_ref[...] * g1_ref[...] + w2_ref[...] * g2_ref[...]


def kernel(x, Wg, w1, w2, w3):
    bsz, seq_len, _ = x.shape
    T = bsz * seq_len
    xf = x.reshape(T, D)

    TM = 256
    nt = T // TM
    i1, i2, w1n, w2n, r1, r2, ce, aux = pl.pallas_call(
        functools.partial(_router_kernel, T=T, TM=TM),
        grid=(nt,),
        in_specs=[
            pl.BlockSpec((TM, D), lambda i: (i, 0)),
            pl.BlockSpec((E, D), lambda i: (0, 0)),
        ],
        out_specs=[pl.BlockSpec((TM, 1), lambda i: (i, 0))] * 6 + [
            pl.BlockSpec((1, E), lambda i: (0, 0)),
            pl.BlockSpec((1, 1), lambda i: (0, 0)),
        ],
        out_shape=[
            jax.ShapeDtypeStruct((T, 1), jnp.int32),
            jax.ShapeDtypeStruct((T, 1), jnp.int32),
            jax.ShapeDtypeStruct((T, 1), jnp.float32),
            jax.ShapeDtypeStruct((T, 1), jnp.float32),
            jax.ShapeDtypeStruct((T, 1), jnp.int32),
            jax.ShapeDtypeStruct((T, 1), jnp.int32),
            jax.ShapeDtypeStruct((1, E), jnp.int32),
            jax.ShapeDtypeStruct((1, 1), jnp.float32),
        ],
        scratch_shapes=[
            pltpu.VMEM((1, E), jnp.float32),
            pltpu.VMEM((1, E), jnp.float32),
        ],
    )(xf, Wg)

    counts = ce[0]
    tiles_per_e = (counts + (TS - 1)) // TS
    tile_start = jnp.concatenate(
        [jnp.zeros((1,), jnp.int32), jnp.cumsum(tiles_per_e)[:-1].astype(jnp.int32)])
    poff = (tile_start * TS).reshape(1, E)
    s_arange = jnp.arange(NT, dtype=jnp.int32)
    tile_expert = (jnp.sum(
        (s_arange[:, None] >= tile_start[None, :]).astype(jnp.int32), axis=1) - 1)

    pos1, pos2, tok = pl.pallas_call(
        functools.partial(_pos_kernel, TM=TM),
        grid=(nt,),
        in_specs=[pl.BlockSpec((TM, 1), lambda i: (i, 0))] * 4 + [
            pl.BlockSpec((1, E), lambda i: (0, 0)),
        ],
        out_specs=[pl.BlockSpec((TM, 1), lambda i: (i, 0))] * 3,
        out_shape=[jax.ShapeDtypeStruct((T, 1), jnp.int32)] * 3,
    )(i1, i2, r1, r2, poff)

    posflat = jnp.concatenate([pos1, pos2], axis=1).reshape(T * K)
    tokflat = jnp.concatenate([tok, tok], axis=1).reshape(T * K)
    xg = _make_dispatch(T)(xf, tokflat, posflat)

    eo = pl.pallas_call(
        _ffn_kernel,
        grid_spec=pltpu.PrefetchScalarGridSpec(
            num_scalar_prefetch=1,
            grid=(NT,),
            in_specs=[
                pl.BlockSpec((TS, D), lambda s, te: (s, 0)),
                pl.BlockSpec((1, F, D), lambda s, te: (te[s], 0, 0)),
                pl.BlockSpec((1, F, D), lambda s, te: (te[s], 0, 0)),
                pl.BlockSpec((1, D, F), lambda s, te: (te[s], 0, 0)),
            ],
            out_specs=pl.BlockSpec((TS, D), lambda s, te: (s, 0)),
        ),
        out_shape=jax.ShapeDtypeStruct((NP, D), jnp.float32),
    )(tile_expert, xg, w1, w3, w2)

    g1, g2 = _make_return(T)(eo, pos1.reshape(T), pos2.reshape(T))

    y = pl.pallas_call(
        _combine_kernel,
        grid=(nt,),
        in_specs=[
            pl.BlockSpec((TM, 1), lambda i: (i, 0)),
            pl.BlockSpec((TM, 1), lambda i: (i, 0)),
            pl.BlockSpec((TM, D), lambda i: (i, 0)),
            pl.BlockSpec((TM, D), lambda i: (i, 0)),
        ],
        out_specs=pl.BlockSpec((TM, D), lambda i: (i, 0)),
        out_shape=jax.ShapeDtypeStruct((T, D), jnp.float32),
    )(w1n, w2n, g1, g2)

    return y.reshape(bsz, seq_len, D), aux[0, 0]


# speedup vs baseline: 1.2496x; 1.2496x over previous
import functools

import jax
import jax.numpy as jnp
from jax import lax
from jax.experimental import pallas as pl
from jax.experimental.pallas import tpu as pltpu
from jax.experimental.pallas import tpu_sc as plsc

E = 8
K = 2
D = 768
F = 2048
ALPHA = 0.1
TS = 256
NP = 4096 + E * TS
NT = NP // TS


def _router_kernel(x_ref, wg_ref, i1_ref, i2_ref, w1_ref, w2_ref,
                   r1_ref, r2_ref, ce_ref, aux_ref, ce_acc, ss_acc, *, T, TM):
    i = pl.program_id(0)
    nt = pl.num_programs(0)
    x = x_ref[...]
    logits = jax.lax.dot_general(
        x, wg_ref[...], (((1,), (1,)), ((), ())),
        preferred_element_type=jnp.float32)
    m = jnp.max(logits, axis=1, keepdims=True)
    ex = jnp.exp(logits - m)
    scores = ex / jnp.sum(ex, axis=1, keepdims=True)

    lane = jax.lax.broadcasted_iota(jnp.int32, scores.shape, 1)
    s1 = jnp.max(scores, axis=1, keepdims=True)
    i1 = jnp.min(jnp.where(scores == s1, lane, E), axis=1, keepdims=True)
    masked = jnp.where(lane == i1, -jnp.inf, scores)
    s2 = jnp.max(masked, axis=1, keepdims=True)
    i2 = jnp.min(jnp.where(masked == s2, lane, E), axis=1, keepdims=True)
    denom = s1 + s2 + 1e-20
    oh1 = (lane == i1).astype(jnp.float32)
    oh2 = (lane == i2).astype(jnp.float32)

    i1_ref[...] = i1
    i2_ref[...] = i2
    w1_ref[...] = s1 / denom
    w2_ref[...] = s2 / denom

    @pl.when(i == 0)
    def _():
        ce_acc[...] = jnp.zeros_like(ce_acc)
        ss_acc[...] = jnp.zeros_like(ss_acc)

    cnt = oh1 + oh2
    row = jax.lax.broadcasted_iota(jnp.int32, (TM, TM), 0)
    col = jax.lax.broadcasted_iota(jnp.int32, (TM, TM), 1)
    lstrict = (col < row).astype(jnp.float32)
    pref = jax.lax.dot_general(
        lstrict, cnt, (((1,), (0,)), ((), ())),
        preferred_element_type=jnp.float32,
        precision=jax.lax.Precision.HIGHEST)
    pref = pref + ce_acc[...]
    r1_ref[...] = jnp.sum(pref * oh1, axis=1, keepdims=True).astype(jnp.int32)
    r2_ref[...] = jnp.sum(pref * oh2, axis=1, keepdims=True).astype(jnp.int32)

    ce_acc[...] += jnp.sum(cnt, axis=0, keepdims=True)
    ss_acc[...] += jnp.sum(scores, axis=0, keepdims=True)

    @pl.when(i == nt - 1)
    def _():
        ce_ref[...] = ce_acc[...].astype(jnp.int32)
        ce = ce_acc[...] / (T * K / E)
        aux_ref[...] = jnp.sum(ce * (ss_acc[...] / T), keepdims=True).reshape(1, 1) * ALPHA


def _pos_kernel(i1_ref, i2_ref, r1_ref, r2_ref, poff_ref, pos1_ref, pos2_ref,
                tok_ref, *, TM):
    i = pl.program_id(0)
    lane = jax.lax.broadcasted_iota(jnp.int32, (TM, E), 1)
    poff = poff_ref[...]
    b1 = jnp.sum(jnp.where(lane == i1_ref[...], poff, 0), axis=1, keepdims=True)
    b2 = jnp.sum(jnp.where(lane == i2_ref[...], poff, 0), axis=1, keepdims=True)
    pos1_ref[...] = b1 + r1_ref[...]
    pos2_ref[...] = b2 + r2_ref[...]
    tok_ref[...] = i * TM + jax.lax.broadcasted_iota(jnp.int32, (TM, 1), 0)


def _make_dispatch(T):
    info = plsc.get_sparse_core_info()
    NC, NS, L = info.num_cores, info.num_subcores, info.num_lanes
    NW = NC * NS
    BW = (T * K) // NW
    mesh = plsc.VectorSubcoreMesh(core_axis_name="c", subcore_axis_name="s")

    @functools.partial(
        pl.kernel, mesh=mesh,
        out_type=jax.ShapeDtypeStruct((NP, D), jnp.float32),
        scratch_types=[
            pltpu.VMEM((BW,), jnp.int32),
            pltpu.VMEM((BW,), jnp.int32),
            pltpu.VMEM((BW, D), jnp.float32),
            pltpu.SemaphoreType.DMA,
        ],
    )
    def dispatch(xf_hbm, tokflat_hbm, posflat_hbm, xg_hbm, tok_v, pos_v, rows_v, sem):
        wid = lax.axis_index("s") * NC + lax.axis_index("c")
        base = wid * BW
        pltpu.sync_copy(tokflat_hbm.at[pl.ds(base, BW)], tok_v)
        pltpu.sync_copy(posflat_hbm.at[pl.ds(base, BW)], pos_v)
        pltpu.async_copy(xf_hbm.at[tok_v], rows_v, sem).wait()
        pltpu.async_copy(rows_v, xg_hbm.at[pos_v], sem).wait()

    return dispatch


def _ffn_kernel(te_ref, xg_ref, w1_ref, w3_ref, w2_ref, eo_ref):
    xg = xg_ref[...]
    h1 = jax.lax.dot_general(
        xg, w1_ref[0], (((1,), (1,)), ((), ())), preferred_element_type=jnp.float32)
    h3 = jax.lax.dot_general(
        xg, w3_ref[0], (((1,), (1,)), ((), ())), preferred_element_type=jnp.float32)
    act = h1 * jax.nn.sigmoid(h1) * h3
    eo_ref[...] = jax.lax.dot_general(
        act, w2_ref[0], (((1,), (1,)), ((), ())), preferred_element_type=jnp.float32)


def _combine_kernel(pos1_ref, pos2_ref, w1_ref, w2_ref, eo_ref, y_ref, *, TM):
    plane = jax.lax.broadcasted_iota(jnp.int32, (TM, NP), 1)
    c = (jnp.where(pos1_ref[...] == plane, w1_ref[...], 0.0)
         + jnp.where(pos2_ref[...] == plane, w2_ref[...], 0.0))
    y_ref[...] = jax.lax.dot_general(
        c, eo_ref[...], (((1,), (0,)), ((), ())),
        preferred_element_type=jnp.float32)


def kernel(x, Wg, w1, w2, w3):
    bsz, seq_len, _ = x.shape
    T = bsz * seq_len
    xf = x.reshape(T, D)

    TM = 256
    nt = T // TM
    i1, i2, w1n, w2n, r1, r2, ce, aux = pl.pallas_call(
        functools.partial(_router_kernel, T=T, TM=TM),
        grid=(nt,),
        in_specs=[
            pl.BlockSpec((TM, D), lambda i: (i, 0)),
            pl.BlockSpec((E, D), lambda i: (0, 0)),
        ],
        out_specs=[pl.BlockSpec((TM, 1), lambda i: (i, 0))] * 6 + [
            pl.BlockSpec((1, E), lambda i: (0, 0)),
            pl.BlockSpec((1, 1), lambda i: (0, 0)),
        ],
        out_shape=[
            jax.ShapeDtypeStruct((T, 1), jnp.int32),
            jax.ShapeDtypeStruct((T, 1), jnp.int32),
            jax.ShapeDtypeStruct((T, 1), jnp.float32),
            jax.ShapeDtypeStruct((T, 1), jnp.float32),
            jax.ShapeDtypeStruct((T, 1), jnp.int32),
            jax.ShapeDtypeStruct((T, 1), jnp.int32),
            jax.ShapeDtypeStruct((1, E), jnp.int32),
            jax.ShapeDtypeStruct((1, 1), jnp.float32),
        ],
        scratch_shapes=[
            pltpu.VMEM((1, E), jnp.float32),
            pltpu.VMEM((1, E), jnp.float32),
        ],
    )(xf, Wg)

    counts = ce[0]
    tiles_per_e = (counts + (TS - 1)) // TS
    tile_start = jnp.concatenate(
        [jnp.zeros((1,), jnp.int32), jnp.cumsum(tiles_per_e)[:-1].astype(jnp.int32)])
    poff = (tile_start * TS).reshape(1, E)
    s_arange = jnp.arange(NT, dtype=jnp.int32)
    tile_expert = (jnp.sum(
        (s_arange[:, None] >= tile_start[None, :]).astype(jnp.int32), axis=1) - 1)

    pos1, pos2, tok = pl.pallas_call(
        functools.partial(_pos_kernel, TM=TM),
        grid=(nt,),
        in_specs=[pl.BlockSpec((TM, 1), lambda i: (i, 0))] * 4 + [
            pl.BlockSpec((1, E), lambda i: (0, 0)),
        ],
        out_specs=[pl.BlockSpec((TM, 1), lambda i: (i, 0))] * 3,
        out_shape=[jax.ShapeDtypeStruct((T, 1), jnp.int32)] * 3,
    )(i1, i2, r1, r2, poff)

    posflat = jnp.concatenate([pos1, pos2], axis=1).reshape(T * K)
    tokflat = jnp.concatenate([tok, tok], axis=1).reshape(T * K)
    xg = _make_dispatch(T)(xf, tokflat, posflat)

    eo = pl.pallas_call(
        _ffn_kernel,
        grid_spec=pltpu.PrefetchScalarGridSpec(
            num_scalar_prefetch=1,
            grid=(NT,),
            in_specs=[
                pl.BlockSpec((TS, D), lambda s, te: (s, 0)),
                pl.BlockSpec((1, F, D), lambda s, te: (te[s], 0, 0)),
                pl.BlockSpec((1, F, D), lambda s, te: (te[s], 0, 0)),
                pl.BlockSpec((1, D, F), lambda s, te: (te[s], 0, 0)),
            ],
            out_specs=pl.BlockSpec((TS, D), lambda s, te: (s, 0)),
        ),
        out_shape=jax.ShapeDtypeStruct((NP, D), jnp.float32),
    )(tile_expert, xg, w1, w3, w2)

    y = pl.pallas_call(
        functools.partial(_combine_kernel, TM=TM),
        grid=(nt,),
        in_specs=[
            pl.BlockSpec((TM, 1), lambda i: (i, 0)),
            pl.BlockSpec((TM, 1), lambda i: (i, 0)),
            pl.BlockSpec((TM, 1), lambda i: (i, 0)),
            pl.BlockSpec((TM, 1), lambda i: (i, 0)),
            pl.BlockSpec((NP, D), lambda i: (0, 0)),
        ],
        out_specs=pl.BlockSpec((TM, D), lambda i: (i, 0)),
        out_shape=jax.ShapeDtypeStruct((T, D), jnp.float32),
    )(pos1, pos2, w1n, w2n, eo)

    return y.reshape(bsz, seq_len, D), aux[0, 0]


# speedup vs baseline: 1.2718x; 1.0178x over previous
import functools

import jax
import jax.numpy as jnp
from jax import lax
from jax.experimental import pallas as pl
from jax.experimental.pallas import tpu as pltpu
from jax.experimental.pallas import tpu_sc as plsc

E = 8
K = 2
D = 768
F = 2048
ALPHA = 0.1
TS = 256
NP = 4096 + E * TS
NT = NP // TS


def _router_kernel(x_ref, wg_ref, i1_ref, i2_ref, w1_ref, w2_ref,
                   r1_ref, r2_ref, ce_ref, aux_ref, ce_acc, ss_acc, *, T, TM):
    i = pl.program_id(0)
    nt = pl.num_programs(0)
    x = x_ref[...]
    logits = jax.lax.dot_general(
        x, wg_ref[...], (((1,), (1,)), ((), ())),
        preferred_element_type=jnp.float32)
    m = jnp.max(logits, axis=1, keepdims=True)
    ex = jnp.exp(logits - m)
    scores = ex / jnp.sum(ex, axis=1, keepdims=True)

    lane = jax.lax.broadcasted_iota(jnp.int32, scores.shape, 1)
    s1 = jnp.max(scores, axis=1, keepdims=True)
    i1 = jnp.min(jnp.where(scores == s1, lane, E), axis=1, keepdims=True)
    masked = jnp.where(lane == i1, -jnp.inf, scores)
    s2 = jnp.max(masked, axis=1, keepdims=True)
    i2 = jnp.min(jnp.where(masked == s2, lane, E), axis=1, keepdims=True)
    denom = s1 + s2 + 1e-20
    oh1 = (lane == i1).astype(jnp.float32)
    oh2 = (lane == i2).astype(jnp.float32)

    i1_ref[...] = i1
    i2_ref[...] = i2
    w1_ref[...] = s1 / denom
    w2_ref[...] = s2 / denom

    @pl.when(i == 0)
    def _():
        ce_acc[...] = jnp.zeros_like(ce_acc)
        ss_acc[...] = jnp.zeros_like(ss_acc)

    cnt = oh1 + oh2
    row = jax.lax.broadcasted_iota(jnp.int32, (TM, TM), 0)
    col = jax.lax.broadcasted_iota(jnp.int32, (TM, TM), 1)
    lstrict = (col < row).astype(jnp.float32)
    pref = jax.lax.dot_general(
        lstrict, cnt, (((1,), (0,)), ((), ())),
        preferred_element_type=jnp.float32,
        precision=jax.lax.Precision.HIGHEST)
    pref = pref + ce_acc[...]
    r1_ref[...] = jnp.sum(pref * oh1, axis=1, keepdims=True).astype(jnp.int32)
    r2_ref[...] = jnp.sum(pref * oh2, axis=1, keepdims=True).astype(jnp.int32)

    ce_acc[...] += jnp.sum(cnt, axis=0, keepdims=True)
    ss_acc[...] += jnp.sum(scores, axis=0, keepdims=True)

    @pl.when(i == nt - 1)
    def _():
        ce_ref[...] = ce_acc[...].astype(jnp.int32)
        ce = ce_acc[...] / (T * K / E)
        aux_ref[...] = jnp.sum(ce * (ss_acc[...] / T), keepdims=True).reshape(1, 1) * ALPHA


def _pos_kernel(i1_ref, i2_ref, r1_ref, r2_ref, poff_ref, pos1_ref, pos2_ref, *, TM):
    lane = jax.lax.broadcasted_iota(jnp.int32, (TM, E), 1)
    poff = poff_ref[...]
    b1 = jnp.sum(jnp.where(lane == i1_ref[...], poff, 0), axis=1, keepdims=True)
    b2 = jnp.sum(jnp.where(lane == i2_ref[...], poff, 0), axis=1, keepdims=True)
    pos1_ref[...] = b1 + r1_ref[...]
    pos2_ref[...] = b2 + r2_ref[...]


def _make_dispatch(T):
    info = plsc.get_sparse_core_info()
    NC, NS, L = info.num_cores, info.num_subcores, info.num_lanes
    NW = NC * NS
    BW = (T * K) // NW
    mesh = plsc.VectorSubcoreMesh(core_axis_name="c", subcore_axis_name="s")

    @functools.partial(
        pl.kernel, mesh=mesh,
        out_type=jax.ShapeDtypeStruct((NP, D), jnp.float32),
        scratch_types=[
            pltpu.VMEM((BW,), jnp.int32),
            pltpu.VMEM((BW,), jnp.int32),
            pltpu.VMEM((BW, D), jnp.float32),
            pltpu.SemaphoreType.DMA,
        ],
    )
    def dispatch(xf_hbm, tokflat_hbm, posflat_hbm, xg_hbm, tok_v, pos_v, rows_v, sem):
        wid = lax.axis_index("s") * NC + lax.axis_index("c")
        base = wid * BW
        pltpu.sync_copy(tokflat_hbm.at[pl.ds(base, BW)], tok_v)
        pltpu.sync_copy(posflat_hbm.at[pl.ds(base, BW)], pos_v)
        pltpu.async_copy(xf_hbm.at[tok_v], rows_v, sem).wait()
        pltpu.async_copy(rows_v, xg_hbm.at[pos_v], sem).wait()

    return dispatch


def _ffn_kernel(te_ref, xg_ref, w1_ref, w3_ref, w2_ref, eo_ref):
    xg = xg_ref[...]
    h1 = jax.lax.dot_general(
        xg, w1_ref[0], (((1,), (1,)), ((), ())), preferred_element_type=jnp.float32)
    h3 = jax.lax.dot_general(
        xg, w3_ref[0], (((1,), (1,)), ((), ())), preferred_element_type=jnp.float32)
    act = h1 * jax.nn.sigmoid(h1) * h3
    eo_ref[...] = jax.lax.dot_general(
        act, w2_ref[0], (((1,), (1,)), ((), ())), preferred_element_type=jnp.float32)


def _combine_kernel(pos1_ref, pos2_ref, w1_ref, w2_ref, eo_ref, y_ref, *, TM):
    plane = jax.lax.broadcasted_iota(jnp.int32, (TM, NP), 1)
    c = (jnp.where(pos1_ref[...] == plane, w1_ref[...], 0.0)
         + jnp.where(pos2_ref[...] == plane, w2_ref[...], 0.0))
    y_ref[...] = jax.lax.dot_general(
        c, eo_ref[...], (((1,), (0,)), ((), ())),
        preferred_element_type=jnp.float32)


def kernel(x, Wg, w1, w2, w3):
    bsz, seq_len, _ = x.shape
    T = bsz * seq_len
    xf = x.reshape(T, D)

    TM = 256
    nt = T // TM
    i1, i2, w1n, w2n, r1, r2, ce, aux = pl.pallas_call(
        functools.partial(_router_kernel, T=T, TM=TM),
        grid=(nt,),
        in_specs=[
            pl.BlockSpec((TM, D), lambda i: (i, 0)),
            pl.BlockSpec((E, D), lambda i: (0, 0)),
        ],
        out_specs=[pl.BlockSpec((TM, 1), lambda i: (i, 0))] * 6 + [
            pl.BlockSpec((1, E), lambda i: (0, 0)),
            pl.BlockSpec((1, 1), lambda i: (0, 0)),
        ],
        out_shape=[
            jax.ShapeDtypeStruct((T, 1), jnp.int32),
            jax.ShapeDtypeStruct((T, 1), jnp.int32),
            jax.ShapeDtypeStruct((T, 1), jnp.float32),
            jax.ShapeDtypeStruct((T, 1), jnp.float32),
            jax.ShapeDtypeStruct((T, 1), jnp.int32),
            jax.ShapeDtypeStruct((T, 1), jnp.int32),
            jax.ShapeDtypeStruct((1, E), jnp.int32),
            jax.ShapeDtypeStruct((1, 1), jnp.float32),
        ],
        scratch_shapes=[
            pltpu.VMEM((1, E), jnp.float32),
            pltpu.VMEM((1, E), jnp.float32),
        ],
    )(xf, Wg)

    counts = ce[0]
    tiles_per_e = (counts + (TS - 1)) // TS
    tile_start = jnp.concatenate(
        [jnp.zeros((1,), jnp.int32), jnp.cumsum(tiles_per_e)[:-1].astype(jnp.int32)])
    poff = (tile_start * TS).reshape(1, E)
    s_arange = jnp.arange(NT, dtype=jnp.int32)
    tile_expert = (jnp.sum(
        (s_arange[:, None] >= tile_start[None, :]).astype(jnp.int32), axis=1) - 1)

    pos1, pos2 = pl.pallas_call(
        functools.partial(_pos_kernel, TM=TM),
        grid=(nt,),
        in_specs=[pl.BlockSpec((TM, 1), lambda i: (i, 0))] * 4 + [
            pl.BlockSpec((1, E), lambda i: (0, 0)),
        ],
        out_specs=[pl.BlockSpec((TM, 1), lambda i: (i, 0))] * 2,
        out_shape=[jax.ShapeDtypeStruct((T, 1), jnp.int32)] * 2,
    )(i1, i2, r1, r2, poff)

    posflat = jnp.concatenate([pos1, pos2], axis=1).reshape(T * K)
    tokflat = jnp.repeat(jnp.arange(T, dtype=jnp.int32), K)
    xg = _make_dispatch(T)(xf, tokflat, posflat)

    eo = pl.pallas_call(
        _ffn_kernel,
        grid_spec=pltpu.PrefetchScalarGridSpec(
            num_scalar_prefetch=1,
            grid=(NT,),
            in_specs=[
                pl.BlockSpec((TS, D), lambda s, te: (s, 0)),
                pl.BlockSpec((1, F, D), lambda s, te: (te[s], 0, 0)),
                pl.BlockSpec((1, F, D), lambda s, te: (te[s], 0, 0)),
                pl.BlockSpec((1, D, F), lambda s, te: (te[s], 0, 0)),
            ],
            out_specs=pl.BlockSpec((TS, D), lambda s, te: (s, 0)),
        ),
        out_shape=jax.ShapeDtypeStruct((NP, D), jnp.float32),
    )(tile_expert, xg, w1, w3, w2)

    y = pl.pallas_call(
        functools.partial(_combine_kernel, TM=TM),
        grid=(nt,),
        in_specs=[
            pl.BlockSpec((TM, 1), lambda i: (i, 0)),
            pl.BlockSpec((TM, 1), lambda i: (i, 0)),
            pl.BlockSpec((TM, 1), lambda i: (i, 0)),
            pl.BlockSpec((TM, 1), lambda i: (i, 0)),
            pl.BlockSpec((NP, D), lambda i: (0, 0)),
        ],
        out_specs=pl.BlockSpec((TM, D), lambda i: (i, 0)),
        out_shape=jax.ShapeDtypeStruct((T, D), jnp.float32),
    )(pos1, pos2, w1n, w2n, eo)

    return y.reshape(bsz, seq_len, D), aux[0, 0]


# speedup vs baseline: 1.2798x; 1.0063x over previous
import functools

import jax
import jax.numpy as jnp
from jax import lax
from jax.experimental import pallas as pl
from jax.experimental.pallas import tpu as pltpu
from jax.experimental.pallas import tpu_sc as plsc

E = 8
K = 2
D = 768
F = 2048
ALPHA = 0.1
TS = 256
NP = 4096 + E * TS
NT = NP // TS


def _router_kernel(x_ref, wg_ref, i1_ref, i2_ref, w1_ref, w2_ref,
                   r1_ref, r2_ref, ce_ref, aux_ref, ce_acc, ss_acc, *, T, TM):
    i = pl.program_id(0)
    nt = pl.num_programs(0)
    x = x_ref[...]
    logits = jax.lax.dot_general(
        x, wg_ref[...], (((1,), (1,)), ((), ())),
        preferred_element_type=jnp.float32)
    m = jnp.max(logits, axis=1, keepdims=True)
    ex = jnp.exp(logits - m)
    scores = ex / jnp.sum(ex, axis=1, keepdims=True)

    lane = jax.lax.broadcasted_iota(jnp.int32, scores.shape, 1)
    s1 = jnp.max(scores, axis=1, keepdims=True)
    i1 = jnp.min(jnp.where(scores == s1, lane, E), axis=1, keepdims=True)
    masked = jnp.where(lane == i1, -jnp.inf, scores)
    s2 = jnp.max(masked, axis=1, keepdims=True)
    i2 = jnp.min(jnp.where(masked == s2, lane, E), axis=1, keepdims=True)
    denom = s1 + s2 + 1e-20
    oh1 = (lane == i1).astype(jnp.float32)
    oh2 = (lane == i2).astype(jnp.float32)

    i1_ref[...] = i1
    i2_ref[...] = i2
    w1_ref[...] = s1 / denom
    w2_ref[...] = s2 / denom

    @pl.when(i == 0)
    def _():
        ce_acc[...] = jnp.zeros_like(ce_acc)
        ss_acc[...] = jnp.zeros_like(ss_acc)

    cnt = oh1 + oh2
    row = jax.lax.broadcasted_iota(jnp.int32, (TM, TM), 0)
    col = jax.lax.broadcasted_iota(jnp.int32, (TM, TM), 1)
    lstrict = (col < row).astype(jnp.float32)
    pref = jax.lax.dot_general(
        lstrict, cnt, (((1,), (0,)), ((), ())),
        preferred_element_type=jnp.float32,
        precision=jax.lax.Precision.HIGHEST)
    pref = pref + ce_acc[...]
    r1_ref[...] = jnp.sum(pref * oh1, axis=1, keepdims=True).astype(jnp.int32)
    r2_ref[...] = jnp.sum(pref * oh2, axis=1, keepdims=True).astype(jnp.int32)

    ce_acc[...] += jnp.sum(cnt, axis=0, keepdims=True)
    ss_acc[...] += jnp.sum(scores, axis=0, keepdims=True)

    @pl.when(i == nt - 1)
    def _():
        ce_ref[...] = ce_acc[...].astype(jnp.int32)
        ce = ce_acc[...] / (T * K / E)
        aux_ref[...] = jnp.sum(ce * (ss_acc[...] / T), keepdims=True).reshape(1, 1) * ALPHA


def _pos_kernel(i1_ref, i2_ref, r1_ref, r2_ref, poff_ref, pos1_ref, pos2_ref,
                posflat_ref, *, TM):
    lane = jax.lax.broadcasted_iota(jnp.int32, (TM, E), 1)
    poff = poff_ref[...]
    b1 = jnp.sum(jnp.where(lane == i1_ref[...], poff, 0), axis=1, keepdims=True)
    b2 = jnp.sum(jnp.where(lane == i2_ref[...], poff, 0), axis=1, keepdims=True)
    p1 = b1 + r1_ref[...]
    p2 = b2 + r2_ref[...]
    pos1_ref[...] = p1
    pos2_ref[...] = p2
    posflat_ref[...] = jnp.concatenate([p1, p2], axis=1)


def _make_dispatch(T):
    info = plsc.get_sparse_core_info()
    NC, NS, L = info.num_cores, info.num_subcores, info.num_lanes
    NW = NC * NS
    BW = (T * K) // NW
    mesh = plsc.VectorSubcoreMesh(core_axis_name="c", subcore_axis_name="s")

    @functools.partial(
        pl.kernel, mesh=mesh,
        out_type=jax.ShapeDtypeStruct((NP, D), jnp.float32),
        scratch_types=[
            pltpu.VMEM((BW,), jnp.int32),
            pltpu.VMEM((BW,), jnp.int32),
            pltpu.VMEM((BW, D), jnp.float32),
            pltpu.SemaphoreType.DMA,
        ],
    )
    def dispatch(xf_hbm, tokflat_hbm, posflat_hbm, xg_hbm, tok_v, pos_v, rows_v, sem):
        wid = lax.axis_index("s") * NC + lax.axis_index("c")
        base = wid * BW
        pltpu.sync_copy(tokflat_hbm.at[pl.ds(base, BW)], tok_v)
        pltpu.sync_copy(posflat_hbm.at[pl.ds(base, BW)], pos_v)
        pltpu.async_copy(xf_hbm.at[tok_v], rows_v, sem).wait()
        pltpu.async_copy(rows_v, xg_hbm.at[pos_v], sem).wait()

    return dispatch


def _ffn_kernel(te_ref, xg_ref, w1_ref, w3_ref, w2_ref, eo_ref):
    xg = xg_ref[...]
    h1 = jax.lax.dot_general(
        xg, w1_ref[0], (((1,), (1,)), ((), ())), preferred_element_type=jnp.float32)
    h3 = jax.lax.dot_general(
        xg, w3_ref[0], (((1,), (1,)), ((), ())), preferred_element_type=jnp.float32)
    act = h1 * jax.nn.sigmoid(h1) * h3
    eo_ref[...] = jax.lax.dot_general(
        act, w2_ref[0], (((1,), (1,)), ((), ())), preferred_element_type=jnp.float32)


def _combine_kernel(pos1_ref, pos2_ref, w1_ref, w2_ref, eo_ref, y_ref, *, TM):
    plane = jax.lax.broadcasted_iota(jnp.int32, (TM, NP), 1)
    c = (jnp.where(pos1_ref[...] == plane, w1_ref[...], 0.0)
         + jnp.where(pos2_ref[...] == plane, w2_ref[...], 0.0))
    y_ref[...] = jax.lax.dot_general(
        c, eo_ref[...], (((1,), (0,)), ((), ())),
        preferred_element_type=jnp.float32)


def kernel(x, Wg, w1, w2, w3):
    bsz, seq_len, _ = x.shape
    T = bsz * seq_len
    xf = x.reshape(T, D)

    TM = 256
    nt = T // TM
    i1, i2, w1n, w2n, r1, r2, ce, aux = pl.pallas_call(
        functools.partial(_router_kernel, T=T, TM=TM),
        grid=(nt,),
        in_specs=[
            pl.BlockSpec((TM, D), lambda i: (i, 0)),
            pl.BlockSpec((E, D), lambda i: (0, 0)),
        ],
        out_specs=[pl.BlockSpec((TM, 1), lambda i: (i, 0))] * 6 + [
            pl.BlockSpec((1, E), lambda i: (0, 0)),
            pl.BlockSpec((1, 1), lambda i: (0, 0)),
        ],
        out_shape=[
            jax.ShapeDtypeStruct((T, 1), jnp.int32),
            jax.ShapeDtypeStruct((T, 1), jnp.int32),
            jax.ShapeDtypeStruct((T, 1), jnp.float32),
            jax.ShapeDtypeStruct((T, 1), jnp.float32),
            jax.ShapeDtypeStruct((T, 1), jnp.int32),
            jax.ShapeDtypeStruct((T, 1), jnp.int32),
            jax.ShapeDtypeStruct((1, E), jnp.int32),
            jax.ShapeDtypeStruct((1, 1), jnp.float32),
        ],
        scratch_shapes=[
            pltpu.VMEM((1, E), jnp.float32),
            pltpu.VMEM((1, E), jnp.float32),
        ],
    )(xf, Wg)

    counts = ce[0]
    tiles_per_e = (counts + (TS - 1)) // TS
    tile_start = jnp.concatenate(
        [jnp.zeros((1,), jnp.int32), jnp.cumsum(tiles_per_e)[:-1].astype(jnp.int32)])
    poff = (tile_start * TS).reshape(1, E)
    s_arange = jnp.arange(NT, dtype=jnp.int32)
    tile_expert = (jnp.sum(
        (s_arange[:, None] >= tile_start[None, :]).astype(jnp.int32), axis=1) - 1)

    pos1, pos2, posflat2 = pl.pallas_call(
        functools.partial(_pos_kernel, TM=TM),
        grid=(nt,),
        in_specs=[pl.BlockSpec((TM, 1), lambda i: (i, 0))] * 4 + [
            pl.BlockSpec((1, E), lambda i: (0, 0)),
        ],
        out_specs=[pl.BlockSpec((TM, 1), lambda i: (i, 0))] * 2 + [
            pl.BlockSpec((TM, K), lambda i: (i, 0)),
        ],
        out_shape=[jax.ShapeDtypeStruct((T, 1), jnp.int32)] * 2 + [
            jax.ShapeDtypeStruct((T, K), jnp.int32),
        ],
    )(i1, i2, r1, r2, poff)

    posflat = posflat2.reshape(T * K)
    tokflat = jnp.repeat(jnp.arange(T, dtype=jnp.int32), K)
    xg = _make_dispatch(T)(xf, tokflat, posflat)

    eo = pl.pallas_call(
        _ffn_kernel,
        grid_spec=pltpu.PrefetchScalarGridSpec(
            num_scalar_prefetch=1,
            grid=(NT,),
            in_specs=[
                pl.BlockSpec((TS, D), lambda s, te: (s, 0)),
                pl.BlockSpec((1, F, D), lambda s, te: (te[s], 0, 0)),
                pl.BlockSpec((1, F, D), lambda s, te: (te[s], 0, 0)),
                pl.BlockSpec((1, D, F), lambda s, te: (te[s], 0, 0)),
            ],
            out_specs=pl.BlockSpec((TS, D), lambda s, te: (s, 0)),
        ),
        out_shape=jax.ShapeDtypeStruct((NP, D), jnp.float32),
    )(tile_expert, xg, w1, w3, w2)

    y = pl.pallas_call(
        functools.partial(_combine_kernel, TM=TM),
        grid=(nt,),
        in_specs=[
            pl.BlockSpec((TM, 1), lambda i: (i, 0)),
            pl.BlockSpec((TM, 1), lambda i: (i, 0)),
            pl.BlockSpec((TM, 1), lambda i: (i, 0)),
            pl.BlockSpec((TM, 1), lambda i: (i, 0)),
            pl.BlockSpec((NP, D), lambda i: (0, 0)),
        ],
        out_specs=pl.BlockSpec((TM, D), lambda i: (i, 0)),
        out_shape=jax.ShapeDtypeStruct((T, D), jnp.float32),
    )(pos1, pos2, w1n, w2n, eo)

    return y.reshape(bsz, seq_len, D), aux[0, 0]


# speedup vs baseline: 1.3418x; 1.0484x over previous
import functools

import jax
import jax.numpy as jnp
from jax import lax
from jax.experimental import pallas as pl
from jax.experimental.pallas import tpu as pltpu
from jax.experimental.pallas import tpu_sc as plsc

E = 8
K = 2
D = 768
F = 2048
ALPHA = 0.1
TS = 256
NP = 4096 + E * TS
NT = NP // TS


def _router_kernel(x_ref, wg_ref, i1_ref, i2_ref, w1_ref, w2_ref,
                   r1_ref, r2_ref, ce_ref, aux_ref, ce_acc, ss_acc, *, T, TM):
    i = pl.program_id(0)
    nt = pl.num_programs(0)
    x = x_ref[...]
    logits = jax.lax.dot_general(
        x, wg_ref[...], (((1,), (1,)), ((), ())),
        preferred_element_type=jnp.float32)
    m = jnp.max(logits, axis=1, keepdims=True)
    ex = jnp.exp(logits - m)
    scores = ex / jnp.sum(ex, axis=1, keepdims=True)

    lane = jax.lax.broadcasted_iota(jnp.int32, scores.shape, 1)
    s1 = jnp.max(scores, axis=1, keepdims=True)
    i1 = jnp.min(jnp.where(scores == s1, lane, E), axis=1, keepdims=True)
    masked = jnp.where(lane == i1, -jnp.inf, scores)
    s2 = jnp.max(masked, axis=1, keepdims=True)
    i2 = jnp.min(jnp.where(masked == s2, lane, E), axis=1, keepdims=True)
    denom = s1 + s2 + 1e-20
    oh1 = (lane == i1).astype(jnp.float32)
    oh2 = (lane == i2).astype(jnp.float32)

    i1_ref[...] = i1
    i2_ref[...] = i2
    w1_ref[...] = s1 / denom
    w2_ref[...] = s2 / denom

    @pl.when(i == 0)
    def _():
        ce_acc[...] = jnp.zeros_like(ce_acc)
        ss_acc[...] = jnp.zeros_like(ss_acc)

    cnt = oh1 + oh2
    row = jax.lax.broadcasted_iota(jnp.int32, (TM, TM), 0)
    col = jax.lax.broadcasted_iota(jnp.int32, (TM, TM), 1)
    lstrict = (col < row).astype(jnp.float32)
    pref = jax.lax.dot_general(
        lstrict, cnt, (((1,), (0,)), ((), ())),
        preferred_element_type=jnp.float32,
        precision=jax.lax.Precision.HIGHEST)
    pref = pref + ce_acc[...]
    r1_ref[...] = jnp.sum(pref * oh1, axis=1, keepdims=True).astype(jnp.int32)
    r2_ref[...] = jnp.sum(pref * oh2, axis=1, keepdims=True).astype(jnp.int32)

    ce_acc[...] += jnp.sum(cnt, axis=0, keepdims=True)
    ss_acc[...] += jnp.sum(scores, axis=0, keepdims=True)

    @pl.when(i == nt - 1)
    def _():
        ce_ref[...] = ce_acc[...].astype(jnp.int32)
        ce = ce_acc[...] / (T * K / E)
        aux_ref[...] = jnp.sum(ce * (ss_acc[...] / T), keepdims=True).reshape(1, 1) * ALPHA


def _pos_kernel(i1_ref, i2_ref, r1_ref, r2_ref, poff_ref, pos1_ref, pos2_ref,
                posflat_ref, *, TM):
    lane = jax.lax.broadcasted_iota(jnp.int32, (TM, E), 1)
    poff = poff_ref[...]
    b1 = jnp.sum(jnp.where(lane == i1_ref[...], poff, 0), axis=1, keepdims=True)
    b2 = jnp.sum(jnp.where(lane == i2_ref[...], poff, 0), axis=1, keepdims=True)
    p1 = b1 + r1_ref[...]
    p2 = b2 + r2_ref[...]
    pos1_ref[...] = p1
    pos2_ref[...] = p2
    posflat_ref[...] = jnp.concatenate([p1, p2], axis=1)


def _make_dispatch(T):
    info = plsc.get_sparse_core_info()
    NC, NS, L = info.num_cores, info.num_subcores, info.num_lanes
    NW = NC * NS
    BW = (T * K) // NW
    mesh = plsc.VectorSubcoreMesh(core_axis_name="c", subcore_axis_name="s")

    @functools.partial(
        pl.kernel, mesh=mesh,
        out_type=jax.ShapeDtypeStruct((NP, D), jnp.float32),
        scratch_types=[
            pltpu.VMEM((BW,), jnp.int32),
            pltpu.VMEM((BW,), jnp.int32),
            pltpu.VMEM((BW, D), jnp.float32),
            pltpu.SemaphoreType.DMA,
        ],
    )
    def dispatch(xf_hbm, tokflat_hbm, posflat_hbm, xg_hbm, tok_v, pos_v, rows_v, sem):
        wid = lax.axis_index("s") * NC + lax.axis_index("c")
        base = wid * BW
        pltpu.sync_copy(tokflat_hbm.at[pl.ds(base, BW)], tok_v)
        pltpu.sync_copy(posflat_hbm.at[pl.ds(base, BW)], pos_v)
        pltpu.async_copy(xf_hbm.at[tok_v], rows_v, sem).wait()
        pltpu.async_copy(rows_v, xg_hbm.at[pos_v], sem).wait()

    return dispatch


def _ffn_kernel(te_ref, xg_ref, w1_ref, w3_ref, w2_ref, eo_ref):
    xg = xg_ref[...]
    h1 = jax.lax.dot_general(
        xg, w1_ref[0], (((1,), (1,)), ((), ())), preferred_element_type=jnp.float32)
    h3 = jax.lax.dot_general(
        xg, w3_ref[0], (((1,), (1,)), ((), ())), preferred_element_type=jnp.float32)
    act = h1 * jax.nn.sigmoid(h1) * h3
    eo_ref[...] = jax.lax.dot_general(
        act, w2_ref[0], (((1,), (1,)), ((), ())), preferred_element_type=jnp.float32)


def _make_return(T):
    info = plsc.get_sparse_core_info()
    NC, NS, _ = info.num_cores, info.num_subcores, info.num_lanes
    NW = NC * NS
    BW = T // NW
    mesh = plsc.VectorSubcoreMesh(core_axis_name="c", subcore_axis_name="s")

    @functools.partial(
        pl.kernel, mesh=mesh,
        out_type=[jax.ShapeDtypeStruct((T, D), jnp.float32),
                  jax.ShapeDtypeStruct((T, D), jnp.float32)],
        scratch_types=[
            pltpu.VMEM((BW,), jnp.int32),
            pltpu.VMEM((BW, D), jnp.float32),
            pltpu.SemaphoreType.DMA,
        ],
    )
    def ret(eo_hbm, pos1_hbm, pos2_hbm, g1_hbm, g2_hbm, idx_v, rows_v, sem):
        wid = lax.axis_index("s") * NC + lax.axis_index("c")
        base = wid * BW
        pltpu.sync_copy(pos1_hbm.at[pl.ds(base, BW)], idx_v)
        pltpu.async_copy(eo_hbm.at[idx_v], rows_v, sem).wait()
        pltpu.sync_copy(rows_v, g1_hbm.at[pl.ds(base, BW)])
        pltpu.sync_copy(pos2_hbm.at[pl.ds(base, BW)], idx_v)
        pltpu.async_copy(eo_hbm.at[idx_v], rows_v, sem).wait()
        pltpu.sync_copy(rows_v, g2_hbm.at[pl.ds(base, BW)])

    return ret


def _combine_kernel(w1_ref, w2_ref, g1_ref, g2_ref, y_ref):
    y_ref[...] = w1_ref[...] * g1_ref[...] + w2_ref[...] * g2_ref[...]


def kernel(x, Wg, w1, w2, w3):
    bsz, seq_len, _ = x.shape
    T = bsz * seq_len
    xf = x.reshape(T, D)

    TM = 256
    nt = T // TM
    i1, i2, w1n, w2n, r1, r2, ce, aux = pl.pallas_call(
        functools.partial(_router_kernel, T=T, TM=TM),
        grid=(nt,),
        in_specs=[
            pl.BlockSpec((TM, D), lambda i: (i, 0)),
            pl.BlockSpec((E, D), lambda i: (0, 0)),
        ],
        out_specs=[pl.BlockSpec((TM, 1), lambda i: (i, 0))] * 6 + [
            pl.BlockSpec((1, E), lambda i: (0, 0)),
            pl.BlockSpec((1, 1), lambda i: (0, 0)),
        ],
        out_shape=[
            jax.ShapeDtypeStruct((T, 1), jnp.int32),
            jax.ShapeDtypeStruct((T, 1), jnp.int32),
            jax.ShapeDtypeStruct((T, 1), jnp.float32),
            jax.ShapeDtypeStruct((T, 1), jnp.float32),
            jax.ShapeDtypeStruct((T, 1), jnp.int32),
            jax.ShapeDtypeStruct((T, 1), jnp.int32),
            jax.ShapeDtypeStruct((1, E), jnp.int32),
            jax.ShapeDtypeStruct((1, 1), jnp.float32),
        ],
        scratch_shapes=[
            pltpu.VMEM((1, E), jnp.float32),
            pltpu.VMEM((1, E), jnp.float32),
        ],
    )(xf, Wg)

    counts = ce[0]
    tiles_per_e = (counts + (TS - 1)) // TS
    tile_start = jnp.concatenate(
        [jnp.zeros((1,), jnp.int32), jnp.cumsum(tiles_per_e)[:-1].astype(jnp.int32)])
    poff = (tile_start * TS).reshape(1, E)
    s_arange = jnp.arange(NT, dtype=jnp.int32)
    tile_expert = (jnp.sum(
        (s_arange[:, None] >= tile_start[None, :]).astype(jnp.int32), axis=1) - 1)

    pos1, pos2, posflat2 = pl.pallas_call(
        functools.partial(_pos_kernel, TM=TM),
        grid=(nt,),
        in_specs=[pl.BlockSpec((TM, 1), lambda i: (i, 0))] * 4 + [
            pl.BlockSpec((1, E), lambda i: (0, 0)),
        ],
        out_specs=[pl.BlockSpec((TM, 1), lambda i: (i, 0))] * 2 + [
            pl.BlockSpec((TM, K), lambda i: (i, 0)),
        ],
        out_shape=[jax.ShapeDtypeStruct((T, 1), jnp.int32)] * 2 + [
            jax.ShapeDtypeStruct((T, K), jnp.int32),
        ],
    )(i1, i2, r1, r2, poff)

    posflat = posflat2.reshape(T * K)
    tokflat = jnp.repeat(jnp.arange(T, dtype=jnp.int32), K)
    xg = _make_dispatch(T)(xf, tokflat, posflat)

    eo = pl.pallas_call(
        _ffn_kernel,
        grid_spec=pltpu.PrefetchScalarGridSpec(
            num_scalar_prefetch=1,
            grid=(NT,),
            in_specs=[
                pl.BlockSpec((TS, D), lambda s, te: (s, 0)),
                pl.BlockSpec((1, F, D), lambda s, te: (te[s], 0, 0)),
                pl.BlockSpec((1, F, D), lambda s, te: (te[s], 0, 0)),
                pl.BlockSpec((1, D, F), lambda s, te: (te[s], 0, 0)),
            ],
            out_specs=pl.BlockSpec((TS, D), lambda s, te: (s, 0)),
        ),
        out_shape=jax.ShapeDtypeStruct((NP, D), jnp.float32),
    )(tile_expert, xg, w1, w3, w2)

    g1, g2 = _make_return(T)(eo, pos1.reshape(T), pos2.reshape(T))

    y = pl.pallas_call(
        _combine_kernel,
        grid=(nt,),
        in_specs=[
            pl.BlockSpec((TM, 1), lambda i: (i, 0)),
            pl.BlockSpec((TM, 1), lambda i: (i, 0)),
            pl.BlockSpec((TM, D), lambda i: (i, 0)),
            pl.BlockSpec((TM, D), lambda i: (i, 0)),
        ],
        out_specs=pl.BlockSpec((TM, D), lambda i: (i, 0)),
        out_shape=jax.ShapeDtypeStruct((T, D), jnp.float32),
    )(w1n, w2n, g1, g2)

    return y.reshape(bsz, seq_len, D), aux[0, 0]


# speedup vs baseline: 1.3938x; 1.0388x over previous
import functools

import jax
import jax.numpy as jnp
from jax import lax
from jax.experimental import pallas as pl
from jax.experimental.pallas import tpu as pltpu
from jax.experimental.pallas import tpu_sc as plsc

E = 8
K = 2
D = 768
F = 2048
ALPHA = 0.1
TS = 256
NP = 4096 + E * TS
NT = NP // TS


def _router_kernel(x_ref, wg_ref, i1_ref, i2_ref, w1_ref, w2_ref,
                   r1_ref, r2_ref, ce_ref, aux_ref, ce_acc, ss_acc, *, T, TM):
    i = pl.program_id(0)
    nt = pl.num_programs(0)
    x = x_ref[...]
    logits = jax.lax.dot_general(
        x, wg_ref[...], (((1,), (1,)), ((), ())),
        preferred_element_type=jnp.float32)
    m = jnp.max(logits, axis=1, keepdims=True)
    ex = jnp.exp(logits - m)
    scores = ex / jnp.sum(ex, axis=1, keepdims=True)

    lane = jax.lax.broadcasted_iota(jnp.int32, scores.shape, 1)
    s1 = jnp.max(scores, axis=1, keepdims=True)
    i1 = jnp.min(jnp.where(scores == s1, lane, E), axis=1, keepdims=True)
    masked = jnp.where(lane == i1, -jnp.inf, scores)
    s2 = jnp.max(masked, axis=1, keepdims=True)
    i2 = jnp.min(jnp.where(masked == s2, lane, E), axis=1, keepdims=True)
    denom = s1 + s2 + 1e-20
    oh1 = (lane == i1).astype(jnp.float32)
    oh2 = (lane == i2).astype(jnp.float32)

    i1_ref[...] = i1
    i2_ref[...] = i2
    w1_ref[...] = s1 / denom
    w2_ref[...] = s2 / denom

    @pl.when(i == 0)
    def _():
        ce_acc[...] = jnp.zeros_like(ce_acc)
        ss_acc[...] = jnp.zeros_like(ss_acc)

    cnt = oh1 + oh2
    row = jax.lax.broadcasted_iota(jnp.int32, (TM, TM), 0)
    col = jax.lax.broadcasted_iota(jnp.int32, (TM, TM), 1)
    lstrict = (col < row).astype(jnp.float32)
    pref = jax.lax.dot_general(
        lstrict, cnt, (((1,), (0,)), ((), ())),
        preferred_element_type=jnp.float32,
        precision=jax.lax.Precision.HIGHEST)
    pref = pref + ce_acc[...]
    r1_ref[...] = jnp.sum(pref * oh1, axis=1, keepdims=True).astype(jnp.int32)
    r2_ref[...] = jnp.sum(pref * oh2, axis=1, keepdims=True).astype(jnp.int32)

    ce_acc[...] += jnp.sum(cnt, axis=0, keepdims=True)
    ss_acc[...] += jnp.sum(scores, axis=0, keepdims=True)

    @pl.when(i == nt - 1)
    def _():
        ce_ref[...] = ce_acc[...].astype(jnp.int32)
        ce = ce_acc[...] / (T * K / E)
        aux_ref[...] = jnp.sum(ce * (ss_acc[...] / T), keepdims=True).reshape(1, 1) * ALPHA


def _pos_kernel(i1_ref, i2_ref, r1_ref, r2_ref, poff_ref, pos1_ref, pos2_ref,
                posflat_ref, *, TM):
    lane = jax.lax.broadcasted_iota(jnp.int32, (TM, E), 1)
    poff = poff_ref[...]
    b1 = jnp.sum(jnp.where(lane == i1_ref[...], poff, 0), axis=1, keepdims=True)
    b2 = jnp.sum(jnp.where(lane == i2_ref[...], poff, 0), axis=1, keepdims=True)
    p1 = b1 + r1_ref[...]
    p2 = b2 + r2_ref[...]
    pos1_ref[...] = p1
    pos2_ref[...] = p2
    posflat_ref[...] = jnp.concatenate([p1, p2], axis=1)


def _make_dispatch(T):
    info = plsc.get_sparse_core_info()
    NC, NS, L = info.num_cores, info.num_subcores, info.num_lanes
    NW = NC * NS
    BW = (T * K) // NW
    mesh = plsc.VectorSubcoreMesh(core_axis_name="c", subcore_axis_name="s")

    @functools.partial(
        pl.kernel, mesh=mesh,
        out_type=jax.ShapeDtypeStruct((NP, D), jnp.float32),
        scratch_types=[
            pltpu.VMEM((BW,), jnp.int32),
            pltpu.VMEM((BW,), jnp.int32),
            pltpu.VMEM((BW, D), jnp.float32),
            pltpu.SemaphoreType.DMA,
        ],
    )
    def dispatch(xf_hbm, tokflat_hbm, posflat_hbm, xg_hbm, tok_v, pos_v, rows_v, sem):
        wid = lax.axis_index("s") * NC + lax.axis_index("c")
        base = wid * BW
        pltpu.sync_copy(tokflat_hbm.at[pl.ds(base, BW)], tok_v)
        pltpu.sync_copy(posflat_hbm.at[pl.ds(base, BW)], pos_v)
        pltpu.async_copy(xf_hbm.at[tok_v], rows_v, sem).wait()
        pltpu.async_copy(rows_v, xg_hbm.at[pos_v], sem).wait()

    return dispatch


def _ffn_kernel(te_ref, xg_ref, w1_ref, w3_ref, w2_ref, eo_ref):
    s = pl.program_id(0)

    @pl.when(s < te_ref[NT])
    def _():
        xg = xg_ref[...]
        h1 = jax.lax.dot_general(
            xg, w1_ref[0], (((1,), (1,)), ((), ())), preferred_element_type=jnp.float32)
        h3 = jax.lax.dot_general(
            xg, w3_ref[0], (((1,), (1,)), ((), ())), preferred_element_type=jnp.float32)
        act = h1 * jax.nn.sigmoid(h1) * h3
        eo_ref[...] = jax.lax.dot_general(
            act, w2_ref[0], (((1,), (1,)), ((), ())), preferred_element_type=jnp.float32)


def _make_return(T):
    info = plsc.get_sparse_core_info()
    NC, NS, _ = info.num_cores, info.num_subcores, info.num_lanes
    NW = NC * NS
    BW = T // NW
    mesh = plsc.VectorSubcoreMesh(core_axis_name="c", subcore_axis_name="s")

    @functools.partial(
        pl.kernel, mesh=mesh,
        out_type=[jax.ShapeDtypeStruct((T, D), jnp.float32),
                  jax.ShapeDtypeStruct((T, D), jnp.float32)],
        scratch_types=[
            pltpu.VMEM((BW,), jnp.int32),
            pltpu.VMEM((BW, D), jnp.float32),
            pltpu.SemaphoreType.DMA,
        ],
    )
    def ret(eo_hbm, pos1_hbm, pos2_hbm, g1_hbm, g2_hbm, idx_v, rows_v, sem):
        wid = lax.axis_index("s") * NC + lax.axis_index("c")
        base = wid * BW
        pltpu.sync_copy(pos1_hbm.at[pl.ds(base, BW)], idx_v)
        pltpu.async_copy(eo_hbm.at[idx_v], rows_v, sem).wait()
        pltpu.sync_copy(rows_v, g1_hbm.at[pl.ds(base, BW)])
        pltpu.sync_copy(pos2_hbm.at[pl.ds(base, BW)], idx_v)
        pltpu.async_copy(eo_hbm.at[idx_v], rows_v, sem).wait()
        pltpu.sync_copy(rows_v, g2_hbm.at[pl.ds(base, BW)])

    return ret


def _combine_kernel(w1_ref, w2_ref, g1_ref, g2_ref, y_ref):
    y_ref[...] = w1_ref[...] * g1_ref[...] + w2_ref[...] * g2_ref[...]


def kernel(x, Wg, w1, w2, w3):
    bsz, seq_len, _ = x.shape
    T = bsz * seq_len
    xf = x.reshape(T, D)

    TM = 256
    nt = T // TM
    i1, i2, w1n, w2n, r1, r2, ce, aux = pl.pallas_call(
        functools.partial(_router_kernel, T=T, TM=TM),
        grid=(nt,),
        in_specs=[
            pl.BlockSpec((TM, D), lambda i: (i, 0)),
            pl.BlockSpec((E, D), lambda i: (0, 0)),
        ],
        out_specs=[pl.BlockSpec((TM, 1), lambda i: (i, 0))] * 6 + [
            pl.BlockSpec((1, E), lambda i: (0, 0)),
            pl.BlockSpec((1, 1), lambda i: (0, 0)),
        ],
        out_shape=[
            jax.ShapeDtypeStruct((T, 1), jnp.int32),
            jax.ShapeDtypeStruct((T, 1), jnp.int32),
            jax.ShapeDtypeStruct((T, 1), jnp.float32),
            jax.ShapeDtypeStruct((T, 1), jnp.float32),
            jax.ShapeDtypeStruct((T, 1), jnp.int32),
            jax.ShapeDtypeStruct((T, 1), jnp.int32),
            jax.ShapeDtypeStruct((1, E), jnp.int32),
            jax.ShapeDtypeStruct((1, 1), jnp.float32),
        ],
        scratch_shapes=[
            pltpu.VMEM((1, E), jnp.float32),
            pltpu.VMEM((1, E), jnp.float32),
        ],
    )(xf, Wg)

    counts = ce[0]
    tiles_per_e = (counts + (TS - 1)) // TS
    tile_start = jnp.concatenate(
        [jnp.zeros((1,), jnp.int32), jnp.cumsum(tiles_per_e)[:-1].astype(jnp.int32)])
    poff = (tile_start * TS).reshape(1, E)
    s_arange = jnp.arange(NT, dtype=jnp.int32)
    tile_expert = (jnp.sum(
        (s_arange[:, None] >= tile_start[None, :]).astype(jnp.int32), axis=1) - 1)
    tiles_used = jnp.sum(tiles_per_e).astype(jnp.int32)
    tile_expert = jnp.concatenate([tile_expert, tiles_used[None]])

    pos1, pos2, posflat2 = pl.pallas_call(
        functools.partial(_pos_kernel, TM=TM),
        grid=(nt,),
        in_specs=[pl.BlockSpec((TM, 1), lambda i: (i, 0))] * 4 + [
            pl.BlockSpec((1, E), lambda i: (0, 0)),
        ],
        out_specs=[pl.BlockSpec((TM, 1), lambda i: (i, 0))] * 2 + [
            pl.BlockSpec((TM, K), lambda i: (i, 0)),
        ],
        out_shape=[jax.ShapeDtypeStruct((T, 1), jnp.int32)] * 2 + [
            jax.ShapeDtypeStruct((T, K), jnp.int32),
        ],
    )(i1, i2, r1, r2, poff)

    posflat = posflat2.reshape(T * K)
    tokflat = jnp.repeat(jnp.arange(T, dtype=jnp.int32), K)
    xg = _make_dispatch(T)(xf, tokflat, posflat)

    eo = pl.pallas_call(
        _ffn_kernel,
        grid_spec=pltpu.PrefetchScalarGridSpec(
            num_scalar_prefetch=1,
            grid=(NT,),
            in_specs=[
                pl.BlockSpec((TS, D), lambda s, te: (s, 0)),
                pl.BlockSpec((1, F, D), lambda s, te: (te[s], 0, 0)),
                pl.BlockSpec((1, F, D), lambda s, te: (te[s], 0, 0)),
                pl.BlockSpec((1, D, F), lambda s, te: (te[s], 0, 0)),
            ],
            out_specs=pl.BlockSpec((TS, D), lambda s, te: (s, 0)),
        ),
        out_shape=jax.ShapeDtypeStruct((NP, D), jnp.float32),
    )(tile_expert, xg, w1, w3, w2)

    g1, g2 = _make_return(T)(eo, pos1.reshape(T), pos2.reshape(T))

    y = pl.pallas_call(
        _combine_kernel,
        grid=(nt,),
        in_specs=[
            pl.BlockSpec((TM, 1), lambda i: (i, 0)),
            pl.BlockSpec((TM, 1), lambda i: (i, 0)),
            pl.BlockSpec((TM, D), lambda i: (i, 0)),
            pl.BlockSpec((TM, D), lambda i: (i, 0)),
        ],
        out_specs=pl.BlockSpec((TM, D), lambda i: (i, 0)),
        out_shape=jax.ShapeDtypeStruct((T, D), jnp.float32),
    )(w1n, w2n, g1, g2)

    return y.reshape(bsz, seq_len, D), aux[0, 0]
